# baseline (speedup 1.0000x reference)
import jax
import jax.numpy as jnp
from jax import lax
from jax.experimental import pallas as pl
from jax.experimental.pallas import tpu as pltpu
from jax.experimental.pallas import tpu_sc as plsc

NUM_CORES = 2
NUM_SUBCORES = 16
NUM_WORKERS = NUM_CORES * NUM_SUBCORES
NUM_CHUNKS = 4
LANES = 16

_DNUMS = lax.GatherDimensionNumbers(
    offset_dims=(), collapsed_slice_dims=(0,), start_index_map=(0,))


def kernel(tokens, embed_table):
    num_tokens = tokens.shape[0]
    vocab, dim = embed_table.shape
    b_per_w = num_tokens // NUM_WORKERS
    chunk = b_per_w // NUM_CHUNKS
    groups_per_chunk = chunk // LANES
    dsub = dim // LANES

    mesh = plsc.VectorSubcoreMesh(core_axis_name="c", subcore_axis_name="s")

    @jax.jit
    def run(tok, table_flat):
        @pl.kernel(
            mesh=mesh,
            out_type=jax.ShapeDtypeStruct((num_tokens * dim,), jnp.float32),
            scratch_types=[
                pltpu.VMEM((b_per_w,), jnp.int32),
                pltpu.VMEM((vocab * dim,), jnp.float32),
                pltpu.VMEM((b_per_w * dim,), jnp.float32),
                pltpu.SemaphoreType.DMA,
            ],
            compiler_params=pltpu.CompilerParams(needs_layout_passes=False),
        )
        def sc_expand(idx_hbm, table_hbm, out_hbm, idx_v, table_v, rows_v,
                      wsem):
            wid = lax.axis_index("s") * NUM_CORES + lax.axis_index("c")
            base = wid * b_per_w
            pltpu.sync_copy(idx_hbm.at[pl.ds(base, b_per_w)], idx_v)
            pltpu.sync_copy(table_hbm, table_v)

            iota = lax.iota(jnp.int32, LANES)
            col_idx = [iota + k * LANES for k in range(dsub)]

            def expand_group(g):
                tok_v = idx_v[pl.ds(g * LANES, LANES)]
                row_base = tok_v * dim
                for j in range(LANES):
                    rb = lax.gather(
                        row_base, jnp.full((LANES, 1), j, jnp.int32), _DNUMS,
                        (1,), mode=lax.GatherScatterMode.PROMISE_IN_BOUNDS)
                    for k in range(dsub):
                        rows_v[pl.ds((g * LANES + j) * dim + k * LANES,
                                     LANES)] = (
                            plsc.load_gather(table_v, [rb + col_idx[k]]))

            writes = []
            for c in range(NUM_CHUNKS):
                plsc.parallel_loop(
                    c * groups_per_chunk, (c + 1) * groups_per_chunk,
                    unroll=2)(expand_group)
                writes.append(pltpu.async_copy(
                    rows_v.at[pl.ds(c * chunk * dim, chunk * dim)],
                    out_hbm.at[pl.ds((base + c * chunk) * dim, chunk * dim)],
                    wsem))
            for w in writes:
                w.wait()

        return sc_expand(tok, table_flat)

    out_flat = run(tokens.astype(jnp.int32), embed_table.reshape(-1))
    return out_flat.reshape(num_tokens, dim)


# speedup vs baseline: 1.1841x; 1.1841x over previous
import jax
import jax.numpy as jnp
from jax import lax
from jax.experimental import pallas as pl
from jax.experimental.pallas import tpu as pltpu
from jax.experimental.pallas import tpu_sc as plsc

NUM_CORES = 2
NUM_SUBCORES = 16
NUM_WORKERS = NUM_CORES * NUM_SUBCORES
LANES = 16
ALU_GROUPS = 18
ALU_CHUNKS = 2
STREAM_SPLIT = 2

_DNUMS = lax.GatherDimensionNumbers(
    offset_dims=(), collapsed_slice_dims=(0,), start_index_map=(0,))


def kernel(tokens, embed_table):
    num_tokens = tokens.shape[0]
    vocab, dim = embed_table.shape
    b_per_w = num_tokens // NUM_WORKERS
    dsub = dim // LANES
    n_alu = ALU_GROUPS * LANES
    n_stream = b_per_w - n_alu
    alu_chunk_groups = ALU_GROUPS // ALU_CHUNKS
    alu_chunk = alu_chunk_groups * LANES
    stream_part = n_stream // STREAM_SPLIT

    mesh = plsc.VectorSubcoreMesh(core_axis_name="c", subcore_axis_name="s")

    @jax.jit
    def run(tok, table2d):
        @pl.kernel(
            mesh=mesh,
            out_type=jax.ShapeDtypeStruct((num_tokens, dim), jnp.float32),
            scratch_types=[
                pltpu.VMEM((b_per_w,), jnp.int32),
                pltpu.VMEM((vocab, dim), jnp.float32),
                pltpu.VMEM((b_per_w, dim), jnp.float32),
                pltpu.SemaphoreType.DMA,
                pltpu.SemaphoreType.DMA,
            ],
            compiler_params=pltpu.CompilerParams(needs_layout_passes=False),
        )
        def sc_expand(idx_hbm, table2d_hbm, out_hbm, idx_v,
                      table_v, rows_v, gsem, wsem):
            wid = lax.axis_index("s") * NUM_CORES + lax.axis_index("c")
            base = wid * b_per_w
            pltpu.sync_copy(idx_hbm.at[pl.ds(base, b_per_w)], idx_v)

            gathers = [
                pltpu.async_copy(
                    table2d_hbm.at[
                        idx_v.at[pl.ds(n_alu + s * stream_part, stream_part)]],
                    rows_v.at[pl.ds(n_alu + s * stream_part, stream_part)],
                    gsem)
                for s in range(STREAM_SPLIT)
            ]

            pltpu.sync_copy(table2d_hbm, table_v)

            iota = lax.iota(jnp.int32, LANES)
            col_idx = [iota + k * LANES for k in range(dsub)]

            def expand_group(g, _):
                tok_v = idx_v[pl.ds(g * LANES, LANES)]
                for j in range(LANES):
                    row = lax.gather(
                        tok_v, jnp.full((LANES, 1), j, jnp.int32), _DNUMS,
                        (1,), mode=lax.GatherScatterMode.PROMISE_IN_BOUNDS)
                    for k in range(dsub):
                        rows_v[g * LANES + j, pl.ds(k * LANES, LANES)] = (
                            plsc.load_gather(table_v, [row, col_idx[k]]))
                return ()

            writes = []
            for c in range(ALU_CHUNKS):
                lax.fori_loop(c * alu_chunk_groups, (c + 1) * alu_chunk_groups,
                              expand_group, (), unroll=False)
                writes.append(pltpu.async_copy(
                    rows_v.at[pl.ds(c * alu_chunk, alu_chunk)],
                    out_hbm.at[pl.ds(base + c * alu_chunk, alu_chunk)], wsem))
            for g in gathers:
                g.wait()
            writes.append(pltpu.async_copy(
                rows_v.at[pl.ds(n_alu, n_stream)],
                out_hbm.at[pl.ds(base + n_alu, n_stream)], wsem))
            for w in writes:
                w.wait()

        return sc_expand(tok, table2d)

    return run(tokens.astype(jnp.int32), embed_table)


# speedup vs baseline: 1.2491x; 1.0549x over previous
import jax
import jax.numpy as jnp
from jax import lax
from jax.experimental import pallas as pl
from jax.experimental.pallas import tpu as pltpu
from jax.experimental.pallas import tpu_sc as plsc

NUM_CORES = 2
NUM_SUBCORES = 16
NUM_WORKERS = NUM_CORES * NUM_SUBCORES
LANES = 16
ALU_GROUPS = 18
ALU_CHUNKS = 2
STREAM_SPLIT = 2

_DNUMS = lax.GatherDimensionNumbers(
    offset_dims=(), collapsed_slice_dims=(0,), start_index_map=(0,))


def kernel(tokens, embed_table):
    num_tokens = tokens.shape[0]
    vocab, dim = embed_table.shape
    b_per_w = num_tokens // NUM_WORKERS
    dsub = dim // LANES
    n_alu = ALU_GROUPS * LANES
    n_stream = b_per_w - n_alu
    alu_chunk_groups = ALU_GROUPS // ALU_CHUNKS
    alu_chunk = alu_chunk_groups * LANES
    stream_part = n_stream // STREAM_SPLIT

    mesh = plsc.VectorSubcoreMesh(core_axis_name="c", subcore_axis_name="s")

    @jax.jit
    def run(tok, table2d):
        @pl.kernel(
            mesh=mesh,
            out_type=jax.ShapeDtypeStruct((num_tokens, dim), jnp.float32),
            scratch_types=[
                pltpu.VMEM((b_per_w,), jnp.int32),
                pltpu.VMEM((vocab, dim), jnp.float32),
                pltpu.VMEM((b_per_w, dim), jnp.float32),
                pltpu.SemaphoreType.DMA,
                pltpu.SemaphoreType.DMA,
            ],
            compiler_params=pltpu.CompilerParams(needs_layout_passes=False),
        )
        def sc_expand(idx_hbm, table2d_hbm, out_hbm, idx_v,
                      table_v, rows_v, gsem, wsem):
            wid = lax.axis_index("s") * NUM_CORES + lax.axis_index("c")
            base = wid * b_per_w
            pltpu.sync_copy(idx_hbm.at[pl.ds(base, b_per_w)], idx_v)

            gathers = [
                pltpu.async_copy(
                    table2d_hbm.at[
                        idx_v.at[pl.ds(n_alu + s * stream_part, stream_part)]],
                    rows_v.at[pl.ds(n_alu + s * stream_part, stream_part)],
                    gsem)
                for s in range(STREAM_SPLIT)
            ]

            pltpu.sync_copy(table2d_hbm, table_v)

            iota = lax.iota(jnp.int32, LANES)
            col_idx = [iota + k * LANES for k in range(dsub)]

            def expand_group(g, _):
                tok_v = idx_v[pl.ds(g * LANES, LANES)]
                for j in range(LANES):
                    row = lax.gather(
                        tok_v, jnp.full((LANES, 1), j, jnp.int32), _DNUMS,
                        (1,), mode=lax.GatherScatterMode.PROMISE_IN_BOUNDS)
                    vals = [plsc.load_gather(table_v, [row, col_idx[k]])
                            for k in range(dsub)]
                    for k in range(dsub):
                        rows_v[g * LANES + j, pl.ds(k * LANES, LANES)] = (
                            vals[k])
                return ()

            writes = []
            for c in range(ALU_CHUNKS):
                lax.fori_loop(c * alu_chunk_groups, (c + 1) * alu_chunk_groups,
                              expand_group, (), unroll=False)
                writes.append(pltpu.async_copy(
                    rows_v.at[pl.ds(c * alu_chunk, alu_chunk)],
                    out_hbm.at[pl.ds(base + c * alu_chunk, alu_chunk)], wsem))
            for g in gathers:
                g.wait()
            writes.append(pltpu.async_copy(
                rows_v.at[pl.ds(n_alu, n_stream)],
                out_hbm.at[pl.ds(base + n_alu, n_stream)], wsem))
            for w in writes:
                w.wait()

        return sc_expand(tok, table2d)

    return run(tokens.astype(jnp.int32), embed_table)
